# baseline (speedup 1.0000x reference)
import jax
import jax.numpy as jnp
from jax import lax
from jax.experimental import pallas as pl
from jax.experimental.pallas import tpu as pltpu
from jax.experimental.pallas import tpu_sc as plsc

_K = 64
_D = 128
_L = 16
_NV = _D // _L
_NC, _NS = 2, 16
_NW = _NC * _NS
_C = 64


def _lane_count_lt(win, t):
    cnt = jnp.int32(0)
    for l in range(_L):
        cnt = cnt + jnp.where(win[l] < t, jnp.int32(1), jnp.int32(0))
    return cnt


def _searchsorted2(idx_hbm, win1, win2, sem1, sem2, t1, t2, num_rows):
    nwin = num_rows // _L
    steps = nwin.bit_length() - 1

    def body(_, st):
        a1, b1, a2, b2 = st
        cont1, cont2 = a1 < b1, a2 < b2
        mid1 = jnp.minimum((a1 + b1) // 2, nwin - 1)
        mid2 = jnp.minimum((a2 + b2) // 2, nwin - 1)
        cp1 = pltpu.async_copy(idx_hbm.at[pl.ds(mid1 * _L, _L)], win1, sem1)
        cp2 = pltpu.async_copy(idx_hbm.at[pl.ds(mid2 * _L, _L)], win2, sem2)
        cp1.wait()
        cp2.wait()
        g1 = win1[...][0]
        g2 = win2[...][0]
        lt1, lt2 = g1 < t1, g2 < t2
        a1n = jnp.where(jnp.logical_and(cont1, lt1), mid1 + 1, a1)
        b1n = jnp.where(jnp.logical_and(cont1, jnp.logical_not(lt1)), mid1, b1)
        a2n = jnp.where(jnp.logical_and(cont2, lt2), mid2 + 1, a2)
        b2n = jnp.where(jnp.logical_and(cont2, jnp.logical_not(lt2)), mid2, b2)
        return a1n, b1n, a2n, b2n

    z, n = jnp.int32(0), jnp.int32(nwin)
    fw1, _, fw2, _ = lax.fori_loop(0, steps, body, (z, n, z, n))
    kw1 = jnp.maximum(fw1 - 1, 0)
    kw2 = jnp.maximum(fw2 - 1, 0)
    cp1 = pltpu.async_copy(idx_hbm.at[pl.ds(kw1 * _L, _L)], win1, sem1)
    cp2 = pltpu.async_copy(idx_hbm.at[pl.ds(kw2 * _L, _L)], win2, sem2)
    cp1.wait()
    cp2.wait()
    r1 = kw1 * _L + _lane_count_lt(win1[...], t1)
    r2 = kw2 * _L + _lane_count_lt(win2[...], t2)
    return r1, r2


def _make_body(num_rows, num_segs):
    segs_w = num_segs // _NW
    last_start = num_rows - _C

    def body(feat_hbm, idx_hbm, out_hbm, fbuf0, fbuf1, ibuf0, ibuf1, acc,
             win1, win2, semA, semB):
        wid = lax.axis_index("c") * _NS + lax.axis_index("s")
        seg_base = wid * segs_w
        lo, hi = _searchsorted2(idx_hbm, win1, win2, semA, semB,
                                seg_base, seg_base + segs_w, num_rows)

        neg = jnp.full((_L,), -jnp.inf, jnp.float32)

        def init_s(s, carry):
            for j in range(_NV):
                acc[s, pl.ds(j * _L, _L)] = neg
            return carry

        lax.fori_loop(0, segs_w, init_s, 0)

        def chunk_start(kc):
            return jnp.minimum(kc * _C, last_start)

        def issue(kc, fbuf, ibuf, sem):
            row0 = chunk_start(kc)
            pltpu.async_copy(feat_hbm.at[pl.ds(row0, _C)], fbuf, sem)
            pltpu.async_copy(idx_hbm.at[pl.ds(row0, _C)],
                             ibuf.at[pl.ds(0, _C)], sem)

        def wait(kc, fbuf, ibuf, sem):
            row0 = chunk_start(kc)
            pltpu.make_async_copy(feat_hbm.at[pl.ds(row0, _C)], fbuf,
                                  sem).wait()
            pltpu.make_async_copy(idx_hbm.at[pl.ds(row0, _C)],
                                  ibuf.at[pl.ds(0, _C)], sem).wait()

        def process(kc, fbuf, ibuf, carry):
            row0 = chunk_start(kc)
            rel0 = jnp.maximum(lo, kc * _C) - row0
            rel1 = jnp.minimum(hi, kc * _C + _C) - row0
            ngroups = jnp.maximum((rel1 - rel0 + _L - 1) // _L, 0)

            def row_body(r, c2):
                cur_seg = c2[0]
                regs = c2[1:]
                seg = ibuf[pl.ds(r, _L)][0]
                changed = seg != cur_seg
                new = tuple(
                    jnp.maximum(jnp.where(changed, neg, regs[j]),
                                fbuf[r, pl.ds(j * _L, _L)])
                    for j in range(_NV)
                )
                for j in range(_NV):
                    acc[seg - seg_base, pl.ds(j * _L, _L)] = new[j]
                return (seg,) + new

            def group_body(gi, c2):
                r = rel0 + _L * gi
                cur_seg = c2[0]
                regs = c2[1:]
                iv = ibuf[pl.ds(r, _L)]
                seg_g = iv[0]
                changed = seg_g != cur_seg
                regs = tuple(jnp.where(changed, neg, regs[j])
                             for j in range(_NV))
                full = jnp.logical_and(seg_g == iv[_L - 1], r + _L <= rel1)

                def fast_body(_, c3):
                    out = []
                    for j in range(_NV):
                        sl = pl.ds(j * _L, _L)
                        p = [fbuf[r + k, sl] for k in range(4)]
                        for k in range(4, _L):
                            p[k % 4] = jnp.maximum(p[k % 4], fbuf[r + k, sl])
                        p01 = jnp.maximum(p[0], p[1])
                        p23 = jnp.maximum(p[2], p[3])
                        out.append(jnp.maximum(c3[j], jnp.maximum(p01, p23)))
                    return tuple(out)

                regs = lax.fori_loop(0, jnp.where(full, 1, 0), fast_body,
                                     regs)
                slow_lo = jnp.where(full, r + _L, r)
                slow_hi = jnp.minimum(r + _L, rel1)
                c4 = lax.fori_loop(slow_lo, jnp.maximum(slow_hi, slow_lo),
                                   row_body, (seg_g,) + regs)
                seg_last = ibuf[pl.ds(slow_hi - 1, _L)][0]
                for j in range(_NV):
                    acc[seg_last - seg_base, pl.ds(j * _L, _L)] = c4[1 + j]
                return c4

            return lax.fori_loop(0, ngroups, group_body, carry)

        kc_lo = lo // _C
        nchunks = jnp.maximum((hi + _C - 1) // _C - kc_lo, 1)
        npairs = (nchunks + 1) // 2

        issue(kc_lo, fbuf0, ibuf0, semA)

        def pair_body(i, carry):
            kca = kc_lo + 2 * i
            kcb = kca + 1
            issue(kcb, fbuf1, ibuf1, semB)
            wait(kca, fbuf0, ibuf0, semA)
            carry = process(kca, fbuf0, ibuf0, carry)
            issue(kca + 2, fbuf0, ibuf0, semA)
            wait(kcb, fbuf1, ibuf1, semB)
            return process(kcb, fbuf1, ibuf1, carry)

        carry0 = (seg_base,) + (neg,) * _NV
        lax.fori_loop(0, npairs, pair_body, carry0)
        wait(kc_lo + 2 * npairs, fbuf0, ibuf0, semA)

        pltpu.sync_copy(acc, out_hbm.at[pl.ds(seg_base, segs_w)])

    return body


def kernel(feature_matrix_batch, cluster_index):
    n, i, d = feature_matrix_batch.shape
    num_rows = n * i
    num_segs = n * _K
    flat = feature_matrix_batch.reshape(num_rows, d)

    mesh = plsc.VectorSubcoreMesh(core_axis_name="c", subcore_axis_name="s")
    run = pl.kernel(
        _make_body(num_rows, num_segs),
        out_type=jax.ShapeDtypeStruct((num_segs, d), jnp.float32),
        mesh=mesh,
        scratch_types=[
            pltpu.VMEM((_C, _D), jnp.float32),
            pltpu.VMEM((_C, _D), jnp.float32),
            pltpu.VMEM((_C + _L,), jnp.int32),
            pltpu.VMEM((_C + _L,), jnp.int32),
            pltpu.VMEM((num_segs // _NW, _D), jnp.float32),
            pltpu.VMEM((_L,), jnp.int32),
            pltpu.VMEM((_L,), jnp.int32),
            pltpu.SemaphoreType.DMA,
            pltpu.SemaphoreType.DMA,
        ],
    )
    out = run(flat, cluster_index)
    return out.reshape(n, _K, d)


# speedup vs baseline: 1.1945x; 1.1945x over previous
import jax
import jax.numpy as jnp
from jax import lax
from jax.experimental import pallas as pl
from jax.experimental.pallas import tpu as pltpu
from jax.experimental.pallas import tpu_sc as plsc

_K = 64
_D = 128
_L = 16
_NV = _D // _L
_NC, _NS = 2, 16
_NW = _NC * _NS
_C = 128


def _lane_count_lt(win, t):
    cnt = jnp.int32(0)
    for l in range(_L):
        cnt = cnt + jnp.where(win[l] < t, jnp.int32(1), jnp.int32(0))
    return cnt


def _searchsorted2(idx_ref, win1, win2, sem1, sem2, t1, t2, num_rows):
    nwin = num_rows // _L
    steps = nwin.bit_length() - 1

    def body(_, st):
        a1, b1, a2, b2 = st
        cont1, cont2 = a1 < b1, a2 < b2
        mid1 = jnp.minimum((a1 + b1) // 2, nwin - 1)
        mid2 = jnp.minimum((a2 + b2) // 2, nwin - 1)
        cp1 = pltpu.async_copy(idx_ref.at[pl.ds(mid1 * _L, _L)], win1, sem1)
        cp2 = pltpu.async_copy(idx_ref.at[pl.ds(mid2 * _L, _L)], win2, sem2)
        cp1.wait()
        cp2.wait()
        g1 = win1[...][0]
        g2 = win2[...][0]
        lt1, lt2 = g1 < t1, g2 < t2
        a1n = jnp.where(jnp.logical_and(cont1, lt1), mid1 + 1, a1)
        b1n = jnp.where(jnp.logical_and(cont1, jnp.logical_not(lt1)), mid1, b1)
        a2n = jnp.where(jnp.logical_and(cont2, lt2), mid2 + 1, a2)
        b2n = jnp.where(jnp.logical_and(cont2, jnp.logical_not(lt2)), mid2, b2)
        return a1n, b1n, a2n, b2n

    z, n = jnp.int32(0), jnp.int32(nwin)
    fw1, _, fw2, _ = lax.fori_loop(0, steps, body, (z, n, z, n))
    kw1 = jnp.maximum(fw1 - 1, 0)
    kw2 = jnp.maximum(fw2 - 1, 0)
    cp1 = pltpu.async_copy(idx_ref.at[pl.ds(kw1 * _L, _L)], win1, sem1)
    cp2 = pltpu.async_copy(idx_ref.at[pl.ds(kw2 * _L, _L)], win2, sem2)
    cp1.wait()
    cp2.wait()
    r1 = kw1 * _L + _lane_count_lt(win1[...], t1)
    r2 = kw2 * _L + _lane_count_lt(win2[...], t2)
    return r1, r2


def _make_body(num_rows, num_segs):
    segs_w = num_segs // _NW
    last_start = num_rows - _C

    def body(feat_hbm, idx_hbm, out_hbm, fbuf0, fbuf1, ibuf0, ibuf1, acc,
             win1, win2, shidx, semA, semB):
        sid = lax.axis_index("s")
        wid = lax.axis_index("c") * _NS + sid
        seg_base = wid * segs_w
        part = num_rows // _NS
        pltpu.async_copy(idx_hbm.at[pl.ds(sid * part, part)],
                         shidx.at[pl.ds(sid * part, part)], semA).wait()
        plsc.subcore_barrier()
        lo, hi = _searchsorted2(shidx, win1, win2, semA, semB,
                                seg_base, seg_base + segs_w, num_rows)

        neg = jnp.full((_L,), -jnp.inf, jnp.float32)

        def init_s(s, carry):
            for j in range(_NV):
                acc[s, pl.ds(j * _L, _L)] = neg
            return carry

        lax.fori_loop(0, segs_w, init_s, 0)

        def chunk_start(kc):
            return jnp.minimum(kc * _C, last_start)

        def issue(kc, fbuf, ibuf, sem):
            row0 = chunk_start(kc)
            pltpu.async_copy(feat_hbm.at[pl.ds(row0, _C)], fbuf, sem)
            pltpu.async_copy(idx_hbm.at[pl.ds(row0, _C)],
                             ibuf.at[pl.ds(0, _C)], sem)

        def wait(kc, fbuf, ibuf, sem):
            row0 = chunk_start(kc)
            pltpu.make_async_copy(feat_hbm.at[pl.ds(row0, _C)], fbuf,
                                  sem).wait()
            pltpu.make_async_copy(idx_hbm.at[pl.ds(row0, _C)],
                                  ibuf.at[pl.ds(0, _C)], sem).wait()

        def process(kc, fbuf, ibuf, carry):
            row0 = chunk_start(kc)
            rel0 = jnp.maximum(lo, kc * _C) - row0
            rel1 = jnp.minimum(hi, kc * _C + _C) - row0
            ngroups = jnp.maximum((rel1 - rel0 + _L - 1) // _L, 0)

            def row_body(r, c2):
                cur_seg = c2[0]
                regs = c2[1:]
                seg = ibuf[pl.ds(r, _L)][0]
                changed = seg != cur_seg
                new = tuple(
                    jnp.maximum(jnp.where(changed, neg, regs[j]),
                                fbuf[r, pl.ds(j * _L, _L)])
                    for j in range(_NV)
                )
                for j in range(_NV):
                    acc[seg - seg_base, pl.ds(j * _L, _L)] = new[j]
                return (seg,) + new

            def group_body(gi, c2):
                r = rel0 + _L * gi
                cur_seg = c2[0]
                regs = c2[1:]
                iv = ibuf[pl.ds(r, _L)]
                seg_g = iv[0]
                changed = seg_g != cur_seg
                regs = tuple(jnp.where(changed, neg, regs[j])
                             for j in range(_NV))
                full = jnp.logical_and(seg_g == iv[_L - 1], r + _L <= rel1)

                def fast_body(_, c3):
                    out = []
                    for j in range(_NV):
                        sl = pl.ds(j * _L, _L)
                        p = [fbuf[r + k, sl] for k in range(4)]
                        for k in range(4, _L):
                            p[k % 4] = jnp.maximum(p[k % 4], fbuf[r + k, sl])
                        p01 = jnp.maximum(p[0], p[1])
                        p23 = jnp.maximum(p[2], p[3])
                        out.append(jnp.maximum(c3[j], jnp.maximum(p01, p23)))
                    return tuple(out)

                regs = lax.fori_loop(0, jnp.where(full, 1, 0), fast_body,
                                     regs)
                slow_lo = jnp.where(full, r + _L, r)
                slow_hi = jnp.minimum(r + _L, rel1)
                c4 = lax.fori_loop(slow_lo, jnp.maximum(slow_hi, slow_lo),
                                   row_body, (seg_g,) + regs)
                seg_last = ibuf[pl.ds(slow_hi - 1, _L)][0]
                for j in range(_NV):
                    acc[seg_last - seg_base, pl.ds(j * _L, _L)] = c4[1 + j]
                return c4

            return lax.fori_loop(0, ngroups, group_body, carry)

        kc_lo = lo // _C
        nchunks = jnp.maximum((hi + _C - 1) // _C - kc_lo, 1)
        npairs = (nchunks + 1) // 2

        issue(kc_lo, fbuf0, ibuf0, semA)

        def pair_body(i, carry):
            kca = kc_lo + 2 * i
            kcb = kca + 1
            issue(kcb, fbuf1, ibuf1, semB)
            wait(kca, fbuf0, ibuf0, semA)
            carry = process(kca, fbuf0, ibuf0, carry)
            issue(kca + 2, fbuf0, ibuf0, semA)
            wait(kcb, fbuf1, ibuf1, semB)
            return process(kcb, fbuf1, ibuf1, carry)

        carry0 = (seg_base,) + (neg,) * _NV
        lax.fori_loop(0, npairs, pair_body, carry0)
        wait(kc_lo + 2 * npairs, fbuf0, ibuf0, semA)

        pltpu.sync_copy(acc, out_hbm.at[pl.ds(seg_base, segs_w)])

    return body


def kernel(feature_matrix_batch, cluster_index):
    n, i, d = feature_matrix_batch.shape
    num_rows = n * i
    num_segs = n * _K
    flat = feature_matrix_batch.reshape(num_rows, d)

    mesh = plsc.VectorSubcoreMesh(core_axis_name="c", subcore_axis_name="s")
    run = pl.kernel(
        _make_body(num_rows, num_segs),
        out_type=jax.ShapeDtypeStruct((num_segs, d), jnp.float32),
        mesh=mesh,
        scratch_types=[
            pltpu.VMEM((_C, _D), jnp.float32),
            pltpu.VMEM((_C, _D), jnp.float32),
            pltpu.VMEM((_C + _L,), jnp.int32),
            pltpu.VMEM((_C + _L,), jnp.int32),
            pltpu.VMEM((num_segs // _NW, _D), jnp.float32),
            pltpu.VMEM((_L,), jnp.int32),
            pltpu.VMEM((_L,), jnp.int32),
            pltpu.VMEM_SHARED((num_rows,), jnp.int32),
            pltpu.SemaphoreType.DMA,
            pltpu.SemaphoreType.DMA,
        ],
    )
    out = run(flat, cluster_index)
    return out.reshape(n, _K, d)
